# Initial kernel scaffold; baseline (speedup 1.0000x reference)
import functools

import jax
import jax.numpy as jnp
from jax import lax
from jax.experimental import pallas as pl
from jax.experimental.pallas import tpu as pltpu
from jax.experimental.pallas import tpu_sc as plsc

L = 16
NC, NS = 2, 16
NW = NC * NS
NSLOT = 65
AGG_W = 528
DEG_W = 80


@functools.lru_cache(maxsize=None)
def _sc_edge_filter(n_nodes: int, n_edges: int, batch: int):
    assert n_edges % NW == 0
    ept = n_edges // NW
    K = 2000
    while ept % K or K % L:
        K //= 2
    n_chunks = ept // K
    assert batch == 64

    mesh = plsc.VectorSubcoreMesh(core_axis_name="c", subcore_axis_name="s")

    @functools.partial(
        pl.kernel,
        mesh=mesh,
        out_type=[
            jax.ShapeDtypeStruct((NW, AGG_W), jnp.float32),
            jax.ShapeDtypeStruct((NW, DEG_W), jnp.float32),
            jax.ShapeDtypeStruct((batch,), jnp.int32),
            jax.ShapeDtypeStruct((batch, 8), jnp.float32),
        ],
        scratch_types=[
            pltpu.VMEM((n_nodes,), jnp.int32),
            pltpu.VMEM((K,), jnp.int32),
            pltpu.VMEM((K,), jnp.int32),
            pltpu.VMEM((K,), jnp.float32),
            pltpu.VMEM((AGG_W,), jnp.float32),
            pltpu.VMEM((DEG_W,), jnp.float32),
            pltpu.VMEM((batch,), jnp.int32),
            pltpu.VMEM((batch,), jnp.int32),
            pltpu.VMEM((L,), jnp.int32),
            pltpu.VMEM((L,), jnp.float32),
            pltpu.VMEM((L,), jnp.int32),
            pltpu.VMEM((L, 8), jnp.float32),
            pltpu.VMEM((batch, 8), jnp.float32),
            pltpu.VMEM((batch,), jnp.int32),
            pltpu.SemaphoreType.DMA,
        ],
    )
    def k(dst_hbm, src_hbm, w_hbm, nf_hbm, bp_hbm, ti_hbm,
          out_agg, out_deg, out_wmap, out_nft,
          table, dstc, srcc, wc, agg, deg, tgt, tis,
          idxs, ws, sls, rows, nfts, wmaps, sem):
        wid = lax.axis_index("s") * NC + lax.axis_index("c")
        iota = lax.iota(jnp.int32, L)
        col = iota & 7
        half = iota >> 3
        zi = jnp.zeros((L,), jnp.int32)
        zf = jnp.zeros((L,), jnp.float32)
        sent = jnp.full((L,), NSLOT - 1, jnp.int32)

        for g in range(AGG_W // L):
            agg[pl.ds(g * L, L)] = zf
        for g in range(DEG_W // L):
            deg[pl.ds(g * L, L)] = zf

        def mset(i, carry):
            table[pl.ds(i * L, L)] = sent
            return carry
        lax.fori_loop(0, n_nodes // L, mset, None)

        pltpu.sync_copy(bp_hbm.at[pl.ds(0, batch)], tgt)
        pltpu.sync_copy(ti_hbm, tis)
        for g in range(batch // L):
            tv = tgt[pl.ds(g * L, L)] + tis[pl.ds(g * L, L)]
            tgt[pl.ds(g * L, L)] = tv
            plsc.store_scatter(table, [tv], iota + g * L)

        base0 = wid * ept

        def chunk_body(c, carry):
            base = base0 + c * K
            pltpu.sync_copy(dst_hbm.at[pl.ds(base, K)], dstc)
            pltpu.sync_copy(src_hbm.at[pl.ds(base, K)], srcc)
            pltpu.sync_copy(w_hbm.at[pl.ds(base, K)], wc)

            def vec_body(i, inner):
                dv = dstc[pl.ds(i * L, L)]
                sl = plsc.load_gather(table, [dv])
                m = sl < (NSLOT - 1)

                @pl.when(jnp.any(m))
                def _process():
                    sv = jnp.where(m, srcc[pl.ds(i * L, L)], zi)
                    wv = jnp.where(m, wc[pl.ds(i * L, L)], zf)
                    slm = jnp.where(m, sl, sent)
                    idxs[...] = sv
                    ws[...] = wv
                    sls[...] = slm
                    plsc.addupdate_scatter(deg, [slm], wv)
                    pltpu.async_copy(nf_hbm.at[idxs], rows, sem).wait()
                    for p in range(8):
                        r = half + 2 * p
                        wb = plsc.load_gather(ws, [r])
                        sb = plsc.load_gather(sls, [r])
                        vals = plsc.load_gather(rows, [r, col])
                        plsc.addupdate_scatter(agg, [sb * 8 + col], vals * wb)

                return inner
            lax.fori_loop(0, K // L, vec_body, None)
            return carry
        lax.fori_loop(0, n_chunks, chunk_body, None)

        pltpu.sync_copy(agg, out_agg.at[wid])
        pltpu.sync_copy(deg, out_deg.at[wid])

        @pl.when(wid == 0)
        def _emit():
            for g in range(batch // L):
                tv = tgt[pl.ds(g * L, L)]
                wmaps[pl.ds(g * L, L)] = plsc.load_gather(table, [tv])
            pltpu.sync_copy(wmaps, out_wmap)
            pltpu.async_copy(nf_hbm.at[tgt], nfts, sem).wait()
            pltpu.sync_copy(nfts, out_nft)

    return k


def _tc_tail_body(pagg, pdeg, wmap, nft, ei, W1, b1, Wa, ba, Wp1, bp1,
                  Wp2, bp2, out):
    agg = jnp.sum(pagg[...], axis=0).reshape(AGG_W // 8, 8)
    deg = jnp.sum(pdeg[...], axis=0)
    b = wmap.shape[0]
    onehot = (wmap[...][:, None]
              == lax.broadcasted_iota(jnp.int32, (b, AGG_W // 8), 1)
              ).astype(jnp.float32)
    agg_b = jnp.dot(onehot, agg, preferred_element_type=jnp.float32)
    deg_b = jnp.dot(onehot, deg[:AGG_W // 8, None],
                    preferred_element_type=jnp.float32)
    x = nft[...][:, :5] + agg_b[:, :5] / (deg_b + 1e-6)
    h = jnp.maximum(
        jnp.dot(x, W1[...], preferred_element_type=jnp.float32) + b1[...], 0.0)
    enr = jnp.maximum(
        jnp.dot(ei[...], Wa[...], preferred_element_type=jnp.float32)
        + ba[...] + h, 0.0)
    hid = jnp.maximum(
        jnp.dot(enr, Wp1[...], preferred_element_type=jnp.float32)
        + bp1[...], 0.0)
    out[...] = jnp.dot(hid, Wp2[...], preferred_element_type=jnp.float32) \
        + bp2[...]


@jax.jit
def kernel(e_i, node_feature, edge_index, edge_weight, batch_ptr, target_idx,
           W1, b1, Wa, ba, Wp1, bp1, Wp2, bp2):
    n, _ = node_feature.shape
    e = edge_weight.shape[0]
    b = e_i.shape[0]
    src = edge_index[0]
    dst = edge_index[1]
    nf_pad = jnp.pad(node_feature, ((0, 0), (0, 3)))

    pagg, pdeg, wmap, nft = _sc_edge_filter(n, e, b)(
        dst, src, edge_weight, nf_pad, batch_ptr, target_idx)

    pred = pl.pallas_call(
        _tc_tail_body,
        out_shape=jax.ShapeDtypeStruct((b, 1), jnp.float32),
    )(pagg, pdeg, wmap, nft, e_i, W1, b1, Wa, ba, Wp1, bp1, Wp2, bp2)
    return pred


# speedup vs baseline: 45.6552x; 45.6552x over previous
import functools

import jax
import jax.numpy as jnp
from jax import lax
from jax.experimental import pallas as pl
from jax.experimental.pallas import tpu as pltpu
from jax.experimental.pallas import tpu_sc as plsc

L = 16
NC, NS = 2, 16
NW = NC * NS
NSLOT = 65
AGG_W = 528
DEG_W = 80


@functools.lru_cache(maxsize=None)
def _sc_edge_filter(n_nodes: int, n_edges: int, batch: int):
    assert n_edges % NW == 0
    ept = n_edges // NW
    K = L
    for cand in range(min(2000, ept), L - 1, -L):
        if ept % cand == 0:
            K = cand
            break
    n_chunks = ept // K
    assert batch == 64

    mesh = plsc.VectorSubcoreMesh(core_axis_name="c", subcore_axis_name="s")

    @functools.partial(
        pl.kernel,
        mesh=mesh,
        compiler_params=pltpu.CompilerParams(
            needs_layout_passes=False, use_tc_tiling_on_sc=False),
        out_type=[
            jax.ShapeDtypeStruct((NW, AGG_W // 8, 8), jnp.float32),
            jax.ShapeDtypeStruct((NW, DEG_W), jnp.float32),
            jax.ShapeDtypeStruct((batch,), jnp.int32),
            jax.ShapeDtypeStruct((batch, 8), jnp.float32),
        ],
        scratch_types=[
            pltpu.VMEM((n_nodes,), jnp.int32),
            pltpu.VMEM((K,), jnp.int32),
            pltpu.VMEM((K,), jnp.int32),
            pltpu.VMEM((K,), jnp.float32),
            pltpu.VMEM((AGG_W // 8, 8), jnp.float32),
            pltpu.VMEM((DEG_W,), jnp.float32),
            pltpu.VMEM((batch,), jnp.int32),
            pltpu.VMEM((batch,), jnp.int32),
            pltpu.VMEM((L,), jnp.int32),
            pltpu.VMEM((L,), jnp.float32),
            pltpu.VMEM((L,), jnp.int32),
            pltpu.VMEM((L, 8), jnp.float32),
            pltpu.VMEM((batch, 8), jnp.float32),
            pltpu.VMEM((batch,), jnp.int32),
            pltpu.SemaphoreType.DMA,
        ],
    )
    def k(dst_hbm, src_hbm, w_hbm, nf_hbm, bp_hbm, ti_hbm,
          out_agg, out_deg, out_wmap, out_nft,
          table, dstc, srcc, wc, agg, deg, tgt, tis,
          idxs, ws, sls, rows, nfts, wmaps, sem):
        wid = lax.axis_index("s") * NC + lax.axis_index("c")
        iota = lax.iota(jnp.int32, L)
        col = iota & 7
        half = iota >> 3
        zi = jnp.zeros((L,), jnp.int32)
        zf = jnp.zeros((L,), jnp.float32)
        sent = jnp.full((L,), NSLOT - 1, jnp.int32)

        for g in range(AGG_W // L):
            flat = iota + g * L
            plsc.store_scatter(agg, [flat >> 3, flat & 7], zf)
        for g in range(DEG_W // L):
            deg[pl.ds(g * L, L)] = zf

        def mset(i, carry):
            table[pl.ds(i * L, L)] = sent
            return carry
        lax.fori_loop(0, n_nodes // L, mset, None)

        pltpu.sync_copy(bp_hbm.at[pl.ds(0, batch)], tgt)
        pltpu.sync_copy(ti_hbm, tis)
        for g in range(batch // L):
            tv = tgt[pl.ds(g * L, L)] + tis[pl.ds(g * L, L)]
            tgt[pl.ds(g * L, L)] = tv
            plsc.store_scatter(table, [tv], iota + g * L)

        base0 = wid * ept

        def chunk_body(c, carry):
            base = base0 + c * K
            pltpu.sync_copy(dst_hbm.at[pl.ds(base, K)], dstc)
            pltpu.sync_copy(src_hbm.at[pl.ds(base, K)], srcc)
            pltpu.sync_copy(w_hbm.at[pl.ds(base, K)], wc)

            def vec_body(i, inner):
                dv = dstc[pl.ds(i * L, L)]
                sl = plsc.load_gather(table, [dv])
                m = sl < (NSLOT - 1)

                @pl.when(jnp.any(m))
                def _process():
                    sv = jnp.where(m, srcc[pl.ds(i * L, L)], zi)
                    wv = jnp.where(m, wc[pl.ds(i * L, L)], zf)
                    slm = jnp.where(m, sl, sent)
                    idxs[...] = sv
                    ws[...] = wv
                    sls[...] = slm
                    plsc.addupdate_scatter(deg, [slm], wv)
                    pltpu.async_copy(nf_hbm.at[idxs], rows, sem).wait()
                    for p in range(8):
                        r = half + 2 * p
                        wb = plsc.load_gather(ws, [r])
                        sb = plsc.load_gather(sls, [r])
                        vals = plsc.load_gather(rows, [r, col])
                        plsc.addupdate_scatter(agg, [sb, col], vals * wb)

                return inner
            lax.fori_loop(0, K // L, vec_body, None)
            return carry
        lax.fori_loop(0, n_chunks, chunk_body, None)

        pltpu.sync_copy(agg, out_agg.at[wid])
        pltpu.sync_copy(deg, out_deg.at[wid])

        @pl.when(wid == 0)
        def _emit():
            for g in range(batch // L):
                tv = tgt[pl.ds(g * L, L)]
                wmaps[pl.ds(g * L, L)] = plsc.load_gather(table, [tv])
            pltpu.sync_copy(wmaps, out_wmap)
            pltpu.async_copy(nf_hbm.at[tgt], nfts, sem).wait()
            pltpu.sync_copy(nfts, out_nft)

    return k


def _tc_tail_body(pagg, pdeg, wmap, nft, ei, W1, b1, Wa, ba, Wp1, bp1,
                  Wp2, bp2, out):
    agg = jnp.sum(pagg[...], axis=0)
    deg = jnp.sum(pdeg[...], axis=0)
    b = wmap.shape[0]
    onehot = (wmap[...][:, None]
              == lax.broadcasted_iota(jnp.int32, (b, AGG_W // 8), 1)
              ).astype(jnp.float32)
    agg_b = jnp.dot(onehot, agg, preferred_element_type=jnp.float32)
    deg_b = jnp.dot(onehot, deg[:AGG_W // 8, None],
                    preferred_element_type=jnp.float32)
    x = nft[...][:, :5] + agg_b[:, :5] / (deg_b + 1e-6)
    h = jnp.maximum(
        jnp.dot(x, W1[...], preferred_element_type=jnp.float32) + b1[...], 0.0)
    enr = jnp.maximum(
        jnp.dot(ei[...], Wa[...], preferred_element_type=jnp.float32)
        + ba[...] + h, 0.0)
    hid = jnp.maximum(
        jnp.dot(enr, Wp1[...], preferred_element_type=jnp.float32)
        + bp1[...], 0.0)
    out[...] = jnp.dot(hid, Wp2[...], preferred_element_type=jnp.float32) \
        + bp2[...]


@jax.jit
def kernel(e_i, node_feature, edge_index, edge_weight, batch_ptr, target_idx,
           W1, b1, Wa, ba, Wp1, bp1, Wp2, bp2):
    n, _ = node_feature.shape
    e = edge_weight.shape[0]
    b = e_i.shape[0]
    src = edge_index[0]
    dst = edge_index[1]
    nf_pad = jnp.pad(node_feature, ((0, 0), (0, 3)))

    pagg, pdeg, wmap, nft = _sc_edge_filter(n, e, b)(
        dst, src, edge_weight, nf_pad, batch_ptr, target_idx)

    pred = pl.pallas_call(
        _tc_tail_body,
        out_shape=jax.ShapeDtypeStruct((b, 1), jnp.float32),
    )(pagg, pdeg, wmap, nft, e_i, W1, b1, Wa, ba, Wp1, bp1, Wp2, bp2)
    return pred


# speedup vs baseline: 125.0603x; 2.7392x over previous
import functools

import jax
import jax.numpy as jnp
from jax import lax
from jax.experimental import pallas as pl
from jax.experimental.pallas import tpu as pltpu
from jax.experimental.pallas import tpu_sc as plsc

L = 16
NC, NS = 2, 16
NW = NC * NS
NROW = 66
G = 5


@functools.lru_cache(maxsize=None)
def _sc_edge_filter(n_nodes: int, n_edges: int, batch: int):
    assert n_edges % NW == 0
    ept = n_edges // NW
    K = L
    for cand in range(min(2000, ept), L - 1, -L):
        if ept % cand == 0:
            K = cand
            break
    n_chunks = ept // K
    assert n_chunks % 2 == 0 and (K // L) % G == 0
    assert batch == 64
    PEND = K + 2 * L

    mesh = plsc.VectorSubcoreMesh(core_axis_name="c", subcore_axis_name="s")

    @functools.partial(
        pl.kernel,
        mesh=mesh,
        compiler_params=pltpu.CompilerParams(
            needs_layout_passes=False, use_tc_tiling_on_sc=False),
        out_type=[
            jax.ShapeDtypeStruct((NW, NROW, 8), jnp.float32),
            jax.ShapeDtypeStruct((batch,), jnp.int32),
            jax.ShapeDtypeStruct((batch, 8), jnp.float32),
        ],
        scratch_types=[
            pltpu.VMEM((n_nodes,), jnp.int32),
            pltpu.VMEM((K,), jnp.int32),
            pltpu.VMEM((K,), jnp.int32),
            pltpu.VMEM((K,), jnp.float32),
            pltpu.VMEM((K,), jnp.int32),
            pltpu.VMEM((K,), jnp.int32),
            pltpu.VMEM((K,), jnp.float32),
            pltpu.VMEM((PEND,), jnp.int32),
            pltpu.VMEM((PEND,), jnp.float32),
            pltpu.VMEM((PEND,), jnp.int32),
            pltpu.VMEM((NROW, 8), jnp.float32),
            pltpu.VMEM((batch,), jnp.int32),
            pltpu.VMEM((batch,), jnp.int32),
            pltpu.VMEM((L,), jnp.int32),
            pltpu.VMEM((L,), jnp.float32),
            pltpu.VMEM((L,), jnp.int32),
            pltpu.VMEM((L, 8), jnp.float32),
            pltpu.VMEM((batch, 8), jnp.float32),
            pltpu.VMEM((batch,), jnp.int32),
            pltpu.SMEM((1,), jnp.int32),
            pltpu.SemaphoreType.DMA,
            pltpu.SemaphoreType.DMA,
            pltpu.SemaphoreType.DMA,
        ],
    )
    def k(dst_hbm, src_hbm, w_hbm, nf_hbm, bp_hbm, ti_hbm,
          out_agg, out_wmap, out_nft,
          table, dstA, srcA, wA, dstB, srcB, wB,
          pend_src, pend_w, pend_sl, agg, tgt, tis,
          idxs, ws, sls, rows, nfts, wmaps, cnt_ref, semA, semB, semG):
        wid = lax.axis_index("s") * NC + lax.axis_index("c")
        iota = lax.iota(jnp.int32, L)
        col = iota & 7
        half = iota >> 3
        m_lo = iota < 8
        m_hi = iota >= 8
        zf = jnp.zeros((L,), jnp.float32)
        zi = jnp.zeros((L,), jnp.int32)
        sent = jnp.full((L,), 64, jnp.int32)

        for g in range(NROW * 8 // L):
            flat = iota + g * L
            plsc.store_scatter(agg, [flat >> 3, flat & 7], zf)

        pltpu.sync_copy(bp_hbm.at[pl.ds(0, batch)], tgt)
        pltpu.sync_copy(ti_hbm, tis)
        for g in range(batch // L):
            tv = tgt[pl.ds(g * L, L)] + tis[pl.ds(g * L, L)]
            tgt[pl.ds(g * L, L)] = tv
            plsc.store_scatter(table, [tv], iota + g * L)
        cnt_ref[0] = 0

        base0 = wid * ept
        bufs = ((dstA, srcA, wA, semA), (dstB, srcB, wB, semB))

        def issue(c, b):
            base = base0 + c * K
            d, s, w, sem = bufs[b]
            pltpu.async_copy(dst_hbm.at[pl.ds(base, K)], d, sem)
            pltpu.async_copy(src_hbm.at[pl.ds(base, K)], s, sem)
            pltpu.async_copy(w_hbm.at[pl.ds(base, K)], w, sem)

        def wait(c, b):
            base = base0 + c * K
            d, s, w, sem = bufs[b]
            pltpu.make_async_copy(dst_hbm.at[pl.ds(base, K)], d, sem).wait()
            pltpu.make_async_copy(src_hbm.at[pl.ds(base, K)], s, sem).wait()
            pltpu.make_async_copy(w_hbm.at[pl.ds(base, K)], w, sem).wait()

        def drain_group(g, carry):
            off = g * L
            idxs[...] = pend_src[pl.ds(off, L)]
            wv = pend_w[pl.ds(off, L)]
            slv = pend_sl[pl.ds(off, L)]
            ws[...] = wv
            sls[...] = slv
            pltpu.async_copy(nf_hbm.at[idxs], rows, semG).wait()
            for p in range(8):
                r = half + 2 * p
                wb = plsc.load_gather(ws, [r])
                sb = plsc.load_gather(sls, [r])
                vals = plsc.load_gather(rows, [r, col])
                contrib = vals * wb
                plsc.addupdate_scatter(agg, [sb, col], contrib, mask=m_lo)
                plsc.addupdate_scatter(agg, [sb, col], contrib, mask=m_hi)
            return carry

        def drain_partial():
            cnt = cnt_ref[0]
            ng = cnt // L

            @pl.when(ng > 0)
            def _():
                lax.fori_loop(0, ng, drain_group, None)
                pend_src[pl.ds(0, L)] = pend_src[pl.ds(ng * L, L)]
                pend_w[pl.ds(0, L)] = pend_w[pl.ds(ng * L, L)]
                pend_sl[pl.ds(0, L)] = pend_sl[pl.ds(ng * L, L)]
                cnt_ref[0] = cnt - ng * L

        def scan(b):
            d, s, w, _ = bufs[b]

            def grp(j, carry):
                i0 = j * G
                dvs, s6s, ms = [], [], []
                for q in range(G):
                    dv = d[pl.ds((i0 + q) * L, L)]
                    s6 = plsc.load_gather(table, [dv]) & 63
                    m = plsc.load_gather(tgt, [s6]) == dv
                    dvs.append(dv)
                    s6s.append(s6)
                    ms.append(m)
                anym = ms[0]
                for q in range(1, G):
                    anym = anym | ms[q]

                @pl.when(jnp.any(anym))
                def _():
                    for q in range(G):
                        m = ms[q]

                        @pl.when(jnp.any(m))
                        def _q():
                            cnt = cnt_ref[0]
                            sv = s[pl.ds((i0 + q) * L, L)]
                            wv = w[pl.ds((i0 + q) * L, L)]
                            plsc.store_compressed(
                                pend_src.at[pl.ds(cnt, L)], sv, mask=m)
                            plsc.store_compressed(
                                pend_w.at[pl.ds(cnt, L)], wv, mask=m)
                            plsc.store_compressed(
                                pend_sl.at[pl.ds(cnt, L)], s6s[q], mask=m)
                            cnt_ref[0] = cnt + jnp.sum(m.astype(jnp.int32))

                return carry
            lax.fori_loop(0, (K // L) // G, grp, None)
            drain_partial()

        issue(0, 0)

        def body(i, carry):
            c = 2 * i
            issue(c + 1, 1)
            wait(c, 0)
            scan(0)

            @pl.when(c + 2 < n_chunks)
            def _():
                issue(c + 2, 0)
            wait(c + 1, 1)
            scan(1)
            return carry
        lax.fori_loop(0, n_chunks // 2, body, None)

        cnt = cnt_ref[0]
        pend_src[pl.ds(cnt, L)] = zi
        pend_w[pl.ds(cnt, L)] = zf
        pend_sl[pl.ds(cnt, L)] = sent
        lax.fori_loop(0, (cnt + L - 1) // L, drain_group, None)

        pltpu.sync_copy(agg, out_agg.at[wid])

        @pl.when(wid == 0)
        def _emit():
            for g in range(batch // L):
                tv = tgt[pl.ds(g * L, L)]
                wmaps[pl.ds(g * L, L)] = plsc.load_gather(table, [tv]) & 63
            pltpu.sync_copy(wmaps, out_wmap)
            pltpu.async_copy(nf_hbm.at[tgt], nfts, semG).wait()
            pltpu.sync_copy(nfts, out_nft)

    return k


def _tc_tail_body(pagg, wmap, nft, ei, W1, b1, Wa, ba, Wp1, bp1,
                  Wp2, bp2, out):
    agg = jnp.sum(pagg[...], axis=0)
    b = wmap.shape[0]
    onehot = (wmap[...][:, None]
              == lax.broadcasted_iota(jnp.int32, (b, NROW), 1)
              ).astype(jnp.float32)
    agg_b = jnp.dot(onehot, agg, preferred_element_type=jnp.float32)
    deg_b = agg_b[:, 5:6]
    x = nft[...][:, :5] + agg_b[:, :5] / (deg_b + 1e-6)
    h = jnp.maximum(
        jnp.dot(x, W1[...], preferred_element_type=jnp.float32) + b1[...], 0.0)
    enr = jnp.maximum(
        jnp.dot(ei[...], Wa[...], preferred_element_type=jnp.float32)
        + ba[...] + h, 0.0)
    hid = jnp.maximum(
        jnp.dot(enr, Wp1[...], preferred_element_type=jnp.float32)
        + bp1[...], 0.0)
    out[...] = jnp.dot(hid, Wp2[...], preferred_element_type=jnp.float32) \
        + bp2[...]


@jax.jit
def kernel(e_i, node_feature, edge_index, edge_weight, batch_ptr, target_idx,
           W1, b1, Wa, ba, Wp1, bp1, Wp2, bp2):
    n, _ = node_feature.shape
    e = edge_weight.shape[0]
    b = e_i.shape[0]
    src = edge_index[0]
    dst = edge_index[1]
    nf_pad = jnp.concatenate(
        [node_feature,
         jnp.ones((n, 1), jnp.float32),
         jnp.zeros((n, 2), jnp.float32)], axis=1)

    pagg, wmap, nft = _sc_edge_filter(n, e, b)(
        dst, src, edge_weight, nf_pad, batch_ptr, target_idx)

    pred = pl.pallas_call(
        _tc_tail_body,
        out_shape=jax.ShapeDtypeStruct((b, 1), jnp.float32),
    )(pagg, wmap, nft, e_i, W1, b1, Wa, ba, Wp1, bp1, Wp2, bp2)
    return pred


# speedup vs baseline: 134.4702x; 1.0752x over previous
import functools

import jax
import jax.numpy as jnp
from jax import lax
from jax.experimental import pallas as pl
from jax.experimental.pallas import tpu as pltpu
from jax.experimental.pallas import tpu_sc as plsc

L = 16
NC, NS = 2, 16
NW = NC * NS
NROW = 66
G = 5


@functools.lru_cache(maxsize=None)
def _sc_edge_filter(n_nodes: int, n_edges: int, batch: int):
    assert n_edges % NW == 0
    ept = n_edges // NW
    K = L
    for cand in range(min(2000, ept), L - 1, -L):
        if ept % cand == 0:
            K = cand
            break
    n_chunks = ept // K
    assert n_chunks % 2 == 0 and (K // L) % G == 0
    assert batch == 64
    PEND = K + 2 * L

    mesh = plsc.VectorSubcoreMesh(core_axis_name="c", subcore_axis_name="s")

    @functools.partial(
        pl.kernel,
        mesh=mesh,
        compiler_params=pltpu.CompilerParams(
            needs_layout_passes=False, use_tc_tiling_on_sc=False),
        out_type=[
            jax.ShapeDtypeStruct((NW, NROW, 8), jnp.float32),
            jax.ShapeDtypeStruct((batch,), jnp.int32),
            jax.ShapeDtypeStruct((batch, 8), jnp.float32),
        ],
        scratch_types=[
            pltpu.VMEM((n_nodes,), jnp.int32),
            pltpu.VMEM((K,), jnp.int32),
            pltpu.VMEM((K,), jnp.int32),
            pltpu.VMEM((K,), jnp.float32),
            pltpu.VMEM((K,), jnp.int32),
            pltpu.VMEM((K,), jnp.int32),
            pltpu.VMEM((K,), jnp.float32),
            pltpu.VMEM((PEND,), jnp.int32),
            pltpu.VMEM((PEND,), jnp.float32),
            pltpu.VMEM((PEND,), jnp.int32),
            pltpu.VMEM((NROW, 8), jnp.float32),
            pltpu.VMEM((batch,), jnp.int32),
            pltpu.VMEM((batch,), jnp.int32),
            pltpu.VMEM((L,), jnp.int32),
            pltpu.VMEM((L,), jnp.float32),
            pltpu.VMEM((L,), jnp.int32),
            pltpu.VMEM((L, 8), jnp.float32),
            pltpu.VMEM((batch, 8), jnp.float32),
            pltpu.VMEM((batch,), jnp.int32),
            pltpu.SMEM((1,), jnp.int32),
            pltpu.SemaphoreType.DMA,
            pltpu.SemaphoreType.DMA,
            pltpu.SemaphoreType.DMA,
        ],
    )
    def k(ei_hbm, w_hbm, nf_hbm, bp_hbm, ti_hbm,
          out_agg, out_wmap, out_nft,
          table, dstA, srcA, wA, dstB, srcB, wB,
          pend_src, pend_w, pend_sl, agg, tgt, tis,
          idxs, ws, sls, rows, nfts, wmaps, cnt_ref, semA, semB, semG):
        wid = lax.axis_index("s") * NC + lax.axis_index("c")
        iota = lax.iota(jnp.int32, L)
        col = iota & 7
        half = iota >> 3
        m_lo = iota < 8
        m_hi = iota >= 8
        zf = jnp.zeros((L,), jnp.float32)
        zi = jnp.zeros((L,), jnp.int32)
        sent = jnp.full((L,), 64, jnp.int32)

        for g in range(NROW * 8 // L):
            flat = iota + g * L
            plsc.store_scatter(agg, [flat >> 3, flat & 7], zf)

        pltpu.sync_copy(bp_hbm.at[pl.ds(0, batch)], tgt)
        pltpu.sync_copy(ti_hbm, tis)
        for g in range(batch // L):
            tv = tgt[pl.ds(g * L, L)] + tis[pl.ds(g * L, L)]
            tgt[pl.ds(g * L, L)] = tv
            plsc.store_scatter(table, [tv], iota + g * L)
        cnt_ref[0] = 0

        base0 = wid * ept
        bufs = ((dstA, srcA, wA, semA), (dstB, srcB, wB, semB))

        def issue(c, b):
            base = base0 + c * K
            d, s, w, sem = bufs[b]
            pltpu.async_copy(ei_hbm.at[1, pl.ds(base, K)], d, sem)
            pltpu.async_copy(ei_hbm.at[0, pl.ds(base, K)], s, sem)
            pltpu.async_copy(w_hbm.at[pl.ds(base, K)], w, sem)

        def wait(c, b):
            base = base0 + c * K
            d, s, w, sem = bufs[b]
            pltpu.make_async_copy(ei_hbm.at[1, pl.ds(base, K)], d, sem).wait()
            pltpu.make_async_copy(ei_hbm.at[0, pl.ds(base, K)], s, sem).wait()
            pltpu.make_async_copy(w_hbm.at[pl.ds(base, K)], w, sem).wait()

        def drain_group(g, carry):
            off = g * L
            idxs[...] = pend_src[pl.ds(off, L)]
            wv = pend_w[pl.ds(off, L)]
            slv = pend_sl[pl.ds(off, L)]
            ws[...] = wv
            sls[...] = slv
            pltpu.async_copy(nf_hbm.at[idxs], rows, semG).wait()
            for p in range(8):
                r = half + 2 * p
                wb = plsc.load_gather(ws, [r])
                sb = plsc.load_gather(sls, [r])
                vals = plsc.load_gather(rows, [r, col])
                contrib = vals * wb
                plsc.addupdate_scatter(agg, [sb, col], contrib, mask=m_lo)
                plsc.addupdate_scatter(agg, [sb, col], contrib, mask=m_hi)
            return carry

        def drain_partial():
            cnt = cnt_ref[0]
            ng = cnt // L

            @pl.when(ng > 0)
            def _():
                lax.fori_loop(0, ng, drain_group, None)
                pend_src[pl.ds(0, L)] = pend_src[pl.ds(ng * L, L)]
                pend_w[pl.ds(0, L)] = pend_w[pl.ds(ng * L, L)]
                pend_sl[pl.ds(0, L)] = pend_sl[pl.ds(ng * L, L)]
                cnt_ref[0] = cnt - ng * L

        def scan(b):
            d, s, w, _ = bufs[b]

            def grp(j, carry):
                i0 = j * G
                dvs, s6s, ms = [], [], []
                for q in range(G):
                    dv = d[pl.ds((i0 + q) * L, L)]
                    s6 = plsc.load_gather(table, [dv]) & 63
                    m = plsc.load_gather(tgt, [s6]) == dv
                    dvs.append(dv)
                    s6s.append(s6)
                    ms.append(m)
                anym = ms[0]
                for q in range(1, G):
                    anym = anym | ms[q]

                @pl.when(jnp.any(anym))
                def _():
                    for q in range(G):
                        m = ms[q]

                        @pl.when(jnp.any(m))
                        def _q():
                            cnt = cnt_ref[0]
                            sv = s[pl.ds((i0 + q) * L, L)]
                            wv = w[pl.ds((i0 + q) * L, L)]
                            plsc.store_compressed(
                                pend_src.at[pl.ds(cnt, L)], sv, mask=m)
                            plsc.store_compressed(
                                pend_w.at[pl.ds(cnt, L)], wv, mask=m)
                            plsc.store_compressed(
                                pend_sl.at[pl.ds(cnt, L)], s6s[q], mask=m)
                            cnt_ref[0] = cnt + jnp.sum(m.astype(jnp.int32))

                return carry
            lax.fori_loop(0, (K // L) // G, grp, None)
            drain_partial()

        issue(0, 0)

        def body(i, carry):
            c = 2 * i
            issue(c + 1, 1)
            wait(c, 0)
            scan(0)

            @pl.when(c + 2 < n_chunks)
            def _():
                issue(c + 2, 0)
            wait(c + 1, 1)
            scan(1)
            return carry
        lax.fori_loop(0, n_chunks // 2, body, None)

        cnt = cnt_ref[0]
        pend_src[pl.ds(cnt, L)] = zi
        pend_w[pl.ds(cnt, L)] = zf
        pend_sl[pl.ds(cnt, L)] = sent
        lax.fori_loop(0, (cnt + L - 1) // L, drain_group, None)

        pltpu.sync_copy(agg, out_agg.at[wid])

        @pl.when(wid == 0)
        def _emit():
            for g in range(batch // L):
                tv = tgt[pl.ds(g * L, L)]
                wmaps[pl.ds(g * L, L)] = plsc.load_gather(table, [tv]) & 63
            pltpu.sync_copy(wmaps, out_wmap)
            pltpu.async_copy(nf_hbm.at[tgt], nfts, semG).wait()
            pltpu.sync_copy(nfts, out_nft)

    return k


def _tc_tail_body(pagg, wmap, nft, ei, W1, b1, Wa, ba, Wp1, bp1,
                  Wp2, bp2, out):
    agg = jnp.sum(pagg[...], axis=0)
    b = wmap.shape[0]
    onehot = (wmap[...][:, None]
              == lax.broadcasted_iota(jnp.int32, (b, NROW), 1)
              ).astype(jnp.float32)
    agg_b = jnp.dot(onehot, agg, preferred_element_type=jnp.float32)
    deg_b = agg_b[:, 5:6]
    x = nft[...][:, :5] + agg_b[:, :5] / (deg_b + 1e-6)
    h = jnp.maximum(
        jnp.dot(x, W1[...], preferred_element_type=jnp.float32) + b1[...], 0.0)
    enr = jnp.maximum(
        jnp.dot(ei[...], Wa[...], preferred_element_type=jnp.float32)
        + ba[...] + h, 0.0)
    hid = jnp.maximum(
        jnp.dot(enr, Wp1[...], preferred_element_type=jnp.float32)
        + bp1[...], 0.0)
    out[...] = jnp.dot(hid, Wp2[...], preferred_element_type=jnp.float32) \
        + bp2[...]


@jax.jit
def kernel(e_i, node_feature, edge_index, edge_weight, batch_ptr, target_idx,
           W1, b1, Wa, ba, Wp1, bp1, Wp2, bp2):
    n, _ = node_feature.shape
    e = edge_weight.shape[0]
    b = e_i.shape[0]
    nf_pad = jnp.concatenate(
        [node_feature,
         jnp.ones((n, 1), jnp.float32),
         jnp.zeros((n, 2), jnp.float32)], axis=1)

    pagg, wmap, nft = _sc_edge_filter(n, e, b)(
        edge_index, edge_weight, nf_pad, batch_ptr, target_idx)

    pred = pl.pallas_call(
        _tc_tail_body,
        out_shape=jax.ShapeDtypeStruct((b, 1), jnp.float32),
    )(pagg, wmap, nft, e_i, W1, b1, Wa, ba, Wp1, bp1, Wp2, bp2)
    return pred
